# baseline (speedup 1.0000x reference)
import jax
import jax.numpy as jnp
from jax.experimental import pallas as pl

EPS = 1e-05
BLOCK_B = 2048


def _body(x_ref, c_ref, w_ref, o_ref):
    x = x_ref[...]
    c = c_ref[...]
    d = 1.0 / (jnp.abs(x - c) + EPS)
    m = jnp.max(d, axis=1, keepdims=True)
    e = jnp.exp(d - m)
    s = jnp.sum(e, axis=1, keepdims=True)
    v = jnp.dot(e, w_ref[...], preferred_element_type=jnp.float32)
    o_ref[...] = v / s


def kernel(x, emb_weight, centroid):
    batch = x.shape[0]
    num_level, embed_dim = emb_weight.shape
    c_row = centroid.reshape(1, num_level)
    grid = batch // BLOCK_B
    return pl.pallas_call(
        _body,
        grid=(grid,),
        in_specs=[
            pl.BlockSpec((BLOCK_B, 1), lambda i: (i, 0)),
            pl.BlockSpec((1, num_level), lambda i: (0, 0)),
            pl.BlockSpec((num_level, embed_dim), lambda i: (0, 0)),
        ],
        out_specs=pl.BlockSpec((BLOCK_B, embed_dim), lambda i: (i, 0)),
        out_shape=jax.ShapeDtypeStruct((batch, embed_dim), jnp.float32),
    )(x, c_row, emb_weight)


# speedup vs baseline: 1.0505x; 1.0505x over previous
import jax
import jax.numpy as jnp
from jax.experimental import pallas as pl

EPS = 1e-05
LOG2E = 1.4426950408889634
BLOCK_B = 4096


def _body(x_ref, c_ref, w_ref, o_ref):
    x = x_ref[...]
    c = c_ref[...]
    d = LOG2E / (jnp.abs(x - c) + EPS)
    m = jnp.max(d, axis=1, keepdims=True)
    e = jnp.exp2(d - m)
    vs = jnp.dot(e, w_ref[...], preferred_element_type=jnp.float32)
    embed_dim = vs.shape[1] - 1
    o_ref[...] = vs[:, :embed_dim] / vs[:, embed_dim:]


def kernel(x, emb_weight, centroid):
    batch = x.shape[0]
    num_level, embed_dim = emb_weight.shape
    c_row = centroid.reshape(1, num_level)
    w_aug = jnp.concatenate(
        [emb_weight, jnp.ones((num_level, 1), jnp.float32)], axis=1)
    grid = batch // BLOCK_B
    return pl.pallas_call(
        _body,
        grid=(grid,),
        in_specs=[
            pl.BlockSpec((BLOCK_B, 1), lambda i: (i, 0)),
            pl.BlockSpec((1, num_level), lambda i: (0, 0)),
            pl.BlockSpec((num_level, embed_dim + 1), lambda i: (0, 0)),
        ],
        out_specs=pl.BlockSpec((BLOCK_B, embed_dim), lambda i: (i, 0)),
        out_shape=jax.ShapeDtypeStruct((batch, embed_dim), jnp.float32),
    )(x, c_row, w_aug)
